# baseline (speedup 1.0000x reference)
import functools

import jax
import jax.numpy as jnp
from jax import lax
from jax.experimental import pallas as pl
from jax.experimental.pallas import tpu as pltpu
from jax.experimental.pallas import tpu_sc as plsc

B = 16384
V = 1000000
D = 32
L = 16
RW = 128
RPF = RW // D

try:
    _info = plsc.get_sparse_core_info()
    NC, NS = _info.num_cores, _info.num_subcores
except ValueError:
    NC, NS = 2, 16
NW = NC * NS
BPW = B // NW
CH = 128
NCH = BPW // CH
HALF = BPW // 2
HCH = HALF // CH
HG = HALF // L

_mesh = plsc.VectorSubcoreMesh(core_axis_name="c", subcore_axis_name="s")


@functools.partial(
    pl.kernel,
    mesh=_mesh,
    out_type=jax.ShapeDtypeStruct((B,), jnp.float32),
    compiler_params=pltpu.CompilerParams(
        needs_layout_passes=False, use_tc_tiling_on_sc=False
    ),
    scratch_types=[
        pltpu.VMEM((NCH, CH), jnp.int32),
        pltpu.VMEM((NCH, CH), jnp.int32),
        pltpu.VMEM((NCH, CH), jnp.int32),
        pltpu.VMEM((NCH, CH), jnp.int32),
        pltpu.VMEM((HALF, RW), jnp.float32),
        pltpu.VMEM((HALF, RW), jnp.float32),
        pltpu.VMEM((BPW,), jnp.float32),
        pltpu.VMEM((BPW,), jnp.float32),
        pltpu.VMEM((BPW,), jnp.float32),
        pltpu.SemaphoreType.DMA,
    ],
)
def _mf_sc(users_hbm, items_hbm, tab_hbm, bu_hbm, bi_hbm, out_hbm,
           uidx, iidx, urow, irow, ud, idt, ubias, ibias, outv, sem):
    wid = lax.axis_index("s") * NC + lax.axis_index("c")
    base = wid * BPW

    for j in range(NCH):
        pltpu.sync_copy(users_hbm.at[pl.ds(base + j * CH, CH)], uidx.at[j])
        pltpu.sync_copy(items_hbm.at[pl.ds(base + j * CH, CH)], iidx.at[j])

    bias_copies = []
    for j in range(NCH):
        sl = pl.ds(j * CH, CH)
        bias_copies.append(
            pltpu.async_copy(bu_hbm.at[uidx.at[j]], ubias.at[sl], sem))
        bias_copies.append(
            pltpu.async_copy(bi_hbm.at[iidx.at[j]], ibias.at[sl], sem))

    for j in range(NCH):
        for c in range(CH // L):
            o = c * L
            urow[j, pl.ds(o, L)] = lax.shift_right_logical(
                uidx[j, pl.ds(o, L)], 2)
            irow[j, pl.ds(o, L)] = lax.shift_right_logical(
                iidx[j, pl.ds(o, L)], 2) + (V * D // RW)

    iota = lax.iota(jnp.int32, L)

    for h in range(2):
        row_copies = []
        for c in range(HCH):
            j = h * HCH + c
            sl = pl.ds(c * CH, CH)
            row_copies.append(
                pltpu.async_copy(tab_hbm.at[urow.at[j]], ud.at[sl], sem))
            row_copies.append(
                pltpu.async_copy(tab_hbm.at[irow.at[j]], idt.at[sl], sem))
        if h == 0:
            for c in bias_copies:
                c.wait()
        for c in row_copies:
            c.wait()

        def group(g, carry):
            j = h * HCH + lax.shift_right_logical(g, 3)
            o = (g & (CH // L - 1)) * L
            gbase = h * HALF + g * L
            bu = uidx[j, pl.ds(o, L)]
            bi = iidx[j, pl.ds(o, L)]
            rows = g * L + iota
            ucol0 = (bu & (RPF - 1)) * D
            icol0 = (bi & (RPF - 1)) * D
            acc = ubias[pl.ds(gbase, L)] + ibias[pl.ds(gbase, L)]
            for d in range(D):
                uv = plsc.load_gather(ud, [rows, ucol0 + d])
                iv = plsc.load_gather(idt, [rows, icol0 + d])
                acc = acc + uv * iv
            outv[pl.ds(gbase, L)] = acc
            return carry

        lax.fori_loop(0, HG, group, 0)

    pltpu.sync_copy(outv, out_hbm.at[pl.ds(base, BPW)])


def kernel(users, items, user_embedding, item_embedding, user_biases, item_biases):
    table = jnp.concatenate(
        [user_embedding.reshape(V * D // RW, RW),
         item_embedding.reshape(V * D // RW, RW)], axis=0)
    return _mf_sc(
        users.astype(jnp.int32),
        items.astype(jnp.int32),
        table,
        user_biases.reshape(-1),
        item_biases.reshape(-1),
    )


# speedup vs baseline: 1.1742x; 1.1742x over previous
import functools

import jax
import jax.numpy as jnp
from jax import lax
from jax.experimental import pallas as pl
from jax.experimental.pallas import tpu as pltpu
from jax.experimental.pallas import tpu_sc as plsc

B = 16384
V = 1000000
D = 32
L = 16
RW = 128
RPF = RW // D

try:
    _info = plsc.get_sparse_core_info()
    NC, NS = _info.num_cores, _info.num_subcores
except ValueError:
    NC, NS = 2, 16
NW = NC * NS
BPW = B // NW
CH = 128
NCH = BPW // CH
HALF = BPW // 2
HCH = HALF // CH
HG = HALF // L

_mesh = plsc.VectorSubcoreMesh(core_axis_name="c", subcore_axis_name="s")


@functools.partial(
    pl.kernel,
    mesh=_mesh,
    out_type=jax.ShapeDtypeStruct((B,), jnp.float32),
    compiler_params=pltpu.CompilerParams(
        needs_layout_passes=False, use_tc_tiling_on_sc=False
    ),
    scratch_types=[
        pltpu.VMEM((NCH, CH), jnp.int32),
        pltpu.VMEM((NCH, CH), jnp.int32),
        pltpu.VMEM((NCH, CH), jnp.int32),
        pltpu.VMEM((NCH, CH), jnp.int32),
        pltpu.VMEM((HALF, RW), jnp.float32),
        pltpu.VMEM((HALF, RW), jnp.float32),
        pltpu.VMEM((BPW,), jnp.float32),
        pltpu.VMEM((BPW,), jnp.float32),
        pltpu.VMEM((BPW,), jnp.float32),
        pltpu.SemaphoreType.DMA,
    ],
)
def _mf_sc(users_hbm, items_hbm, ue_hbm, ie_hbm, bu_hbm, bi_hbm, out_hbm,
           uidx, iidx, urow, irow, ud, idt, ubias, ibias, outv, sem):
    wid = lax.axis_index("s") * NC + lax.axis_index("c")
    base = wid * BPW

    for j in range(NCH):
        pltpu.sync_copy(users_hbm.at[pl.ds(base + j * CH, CH)], uidx.at[j])
        pltpu.sync_copy(items_hbm.at[pl.ds(base + j * CH, CH)], iidx.at[j])

    bias_copies = []
    for j in range(NCH):
        sl = pl.ds(j * CH, CH)
        bias_copies.append(
            pltpu.async_copy(bu_hbm.at[uidx.at[j]], ubias.at[sl], sem))
        bias_copies.append(
            pltpu.async_copy(bi_hbm.at[iidx.at[j]], ibias.at[sl], sem))

    for j in range(NCH):
        for c in range(CH // L):
            o = c * L
            urow[j, pl.ds(o, L)] = lax.shift_right_logical(
                uidx[j, pl.ds(o, L)], 2)
            irow[j, pl.ds(o, L)] = lax.shift_right_logical(
                iidx[j, pl.ds(o, L)], 2)

    iota = lax.iota(jnp.int32, L)

    for h in range(2):
        row_copies = []
        for c in range(HCH):
            j = h * HCH + c
            sl = pl.ds(c * CH, CH)
            row_copies.append(
                pltpu.async_copy(ue_hbm.at[urow.at[j]], ud.at[sl], sem))
            row_copies.append(
                pltpu.async_copy(ie_hbm.at[irow.at[j]], idt.at[sl], sem))
        if h == 0:
            for c in bias_copies:
                c.wait()
        for c in row_copies:
            c.wait()

        def group(g, carry):
            j = h * HCH + lax.shift_right_logical(g, 3)
            o = (g & (CH // L - 1)) * L
            gbase = h * HALF + g * L
            bu = uidx[j, pl.ds(o, L)]
            bi = iidx[j, pl.ds(o, L)]
            rows = g * L + iota
            ucol0 = (bu & (RPF - 1)) * D
            icol0 = (bi & (RPF - 1)) * D
            acc = ubias[pl.ds(gbase, L)] + ibias[pl.ds(gbase, L)]
            for d in range(D):
                uv = plsc.load_gather(ud, [rows, ucol0 + d])
                iv = plsc.load_gather(idt, [rows, icol0 + d])
                acc = acc + uv * iv
            outv[pl.ds(gbase, L)] = acc
            return carry

        lax.fori_loop(0, HG, group, 0)

    pltpu.sync_copy(outv, out_hbm.at[pl.ds(base, BPW)])


def kernel(users, items, user_embedding, item_embedding, user_biases, item_biases):
    return _mf_sc(
        users.astype(jnp.int32),
        items.astype(jnp.int32),
        user_embedding.reshape(V * D // RW, RW),
        item_embedding.reshape(V * D // RW, RW),
        user_biases.reshape(-1),
        item_biases.reshape(-1),
    )
